# Initial kernel scaffold; baseline (speedup 1.0000x reference)
import jax
import jax.numpy as jnp
from jax.experimental import pallas as pl


def kernel(inputs, rel_embeddings):
    raise NotImplementedError("write your pallas kernel here")


# speedup vs baseline: 6.1344x; 6.1344x over previous
import functools

import jax
import jax.numpy as jnp
from jax import lax
from jax.experimental import pallas as pl
from jax.experimental.pallas import tpu as pltpu
from jax.experimental.pallas import tpu_sc as plsc

MAX_POSITION = 2048
DEPTH = 64


@functools.partial(jax.jit, static_argnums=(1,))
def _rpe_expand(table, length):
    L = length
    D = table.shape[-1]
    info = plsc.get_sparse_core_info()
    nc, ns = info.num_cores, info.num_subcores
    nw = nc * ns
    rpw = L // nw
    win = L + rpw

    mesh = plsc.VectorSubcoreMesh(core_axis_name="c", subcore_axis_name="s")

    @functools.partial(
        pl.kernel,
        mesh=mesh,
        out_type=jax.ShapeDtypeStruct((L, L, D), jnp.float32),
        scratch_types=[pltpu.VMEM((win, D), jnp.float32)],
        compiler_params=pltpu.CompilerParams(use_tc_tiling_on_sc=False),
    )
    def k(table_hbm, out_hbm, buf):
        wid = lax.axis_index("s") * nc + lax.axis_index("c")
        i0 = wid * rpw
        pltpu.sync_copy(table_hbm.at[pl.ds(i0 + L, win)], buf)

        def rev_body(p, carry):
            q = win - 1 - p
            for c in range(D // 16):
                s = pl.ds(c * 16, 16)
                a = buf[p, s]
                b = buf[q, s]
                buf[p, s] = b
                buf[q, s] = a
            return carry
        lax.fori_loop(0, win // 2, rev_body, 0)

        def out_body(t, carry):
            pltpu.sync_copy(buf.at[pl.ds(rpw - t, L)], out_hbm.at[i0 + t])
            return carry
        lax.fori_loop(0, rpw, out_body, 0)

    return k(table)


def kernel(inputs, rel_embeddings):
    return _rpe_expand(rel_embeddings, inputs.shape[1])
